# baseline (speedup 1.0000x reference)
import jax
import jax.numpy as jnp
from jax.experimental import pallas as pl
from jax.experimental.pallas import tpu as pltpu

_EPS = 1e-5


def _fused_mlp_kernel(x_ref, m_ref, W1_ref, b1_ref, g1_ref, be1_ref,
                      W2_ref, b2_ref, g2_ref, be2_ref, W3_ref, b3_ref,
                      out_ref,
                      s1_ref, q1_ref, sa1_ref, S_ref, cnt_ref,
                      sc1_ref, sh1_ref, sc2_ref, sh2_ref):
    phase = pl.program_id(0)
    i = pl.program_id(1)

    @pl.when((phase == 0) & (i == 0))
    def _init():
        s1_ref[...] = jnp.zeros_like(s1_ref)
        q1_ref[...] = jnp.zeros_like(q1_ref)
        sa1_ref[...] = jnp.zeros_like(sa1_ref)
        S_ref[...] = jnp.zeros_like(S_ref)
        cnt_ref[0, 0] = 0.0

    x = x_ref[...]
    m = m_ref[...]
    h1 = jnp.dot(x, W1_ref[...], preferred_element_type=jnp.float32) + b1_ref[...]

    @pl.when(phase == 0)
    def _p0():
        hm = h1 * m
        s1_ref[...] += jnp.sum(hm, axis=0, keepdims=True)
        q1_ref[...] += jnp.sum(hm * h1, axis=0, keepdims=True)
        cnt_ref[0, 0] += jnp.sum(m)

    @pl.when((phase == 1) & (i == 0))
    def _bn1_params():
        c = jnp.maximum(cnt_ref[0, 0], 1.0)
        mean = s1_ref[...] / c
        var = q1_ref[...] / c - mean * mean
        sc = g1_ref[...] * jax.lax.rsqrt(var + _EPS)
        sc1_ref[...] = sc
        sh1_ref[...] = be1_ref[...] - mean * sc

    @pl.when(phase >= 1)
    def _p12():
        a1 = jnp.maximum(h1 * sc1_ref[...] + sh1_ref[...], 0.0)

        @pl.when(phase == 1)
        def _p1():
            a1m = a1 * m
            sa1_ref[...] += jnp.sum(a1m, axis=0, keepdims=True)
            S_ref[...] += jax.lax.dot_general(
                a1m, a1, (((0,), (0,)), ((), ())),
                preferred_element_type=jnp.float32)

        @pl.when(phase == 2)
        def _p2():
            @pl.when(i == 0)
            def _bn2_params():
                c = jnp.maximum(cnt_ref[0, 0], 1.0)
                W2v = W2_ref[...]
                b2v = b2_ref[...]
                sh2v = jnp.dot(sa1_ref[...], W2v,
                               preferred_element_type=jnp.float32)
                T = jnp.dot(S_ref[...], W2v,
                            preferred_element_type=jnp.float32)
                q2 = (jnp.sum(W2v * T, axis=0, keepdims=True)
                      + 2.0 * b2v * sh2v + c * b2v * b2v)
                mean = sh2v / c + b2v
                var = q2 / c - mean * mean
                sc = g2_ref[...] * jax.lax.rsqrt(var + _EPS)
                sc2_ref[...] = sc
                sh2_ref[...] = be2_ref[...] - mean * sc

            h2 = jnp.dot(a1, W2_ref[...], preferred_element_type=jnp.float32) + b2_ref[...]
            a2 = jnp.maximum(h2 * sc2_ref[...] + sh2_ref[...], 0.0)
            y = jnp.dot(a2, W3_ref[...], preferred_element_type=jnp.float32) + b3_ref[...]
            out_ref[...] = y * m


def _fused_mlp(x, m, W1, b1, g1, be1, W2, b2, g2, be2, W3, b3, blk):
    R, IN = x.shape
    H1 = W1.shape[1]
    H2 = W2.shape[1]
    OUTD = W3.shape[1]
    nb = R // blk

    def rows(p, i):
        return (i, 0)

    def whole(p, i):
        return (0, 0)

    out = pl.pallas_call(
        _fused_mlp_kernel,
        grid=(3, nb),
        in_specs=[
            pl.BlockSpec((blk, IN), rows),
            pl.BlockSpec((blk, 1), rows),
            pl.BlockSpec((IN, H1), whole),
            pl.BlockSpec((1, H1), whole),
            pl.BlockSpec((1, H1), whole),
            pl.BlockSpec((1, H1), whole),
            pl.BlockSpec((H1, H2), whole),
            pl.BlockSpec((1, H2), whole),
            pl.BlockSpec((1, H2), whole),
            pl.BlockSpec((1, H2), whole),
            pl.BlockSpec((H2, OUTD), whole),
            pl.BlockSpec((1, OUTD), whole),
        ],
        out_specs=pl.BlockSpec((blk, OUTD), lambda p, i: (jnp.where(p == 2, i, 0), 0)),
        out_shape=jax.ShapeDtypeStruct((R, OUTD), jnp.float32),
        scratch_shapes=[
            pltpu.VMEM((1, H1), jnp.float32),
            pltpu.VMEM((1, H1), jnp.float32),
            pltpu.VMEM((1, H1), jnp.float32),
            pltpu.VMEM((H1, H1), jnp.float32),
            pltpu.SMEM((1, 1), jnp.float32),
            pltpu.VMEM((1, H1), jnp.float32),
            pltpu.VMEM((1, H1), jnp.float32),
            pltpu.VMEM((1, H2), jnp.float32),
            pltpu.VMEM((1, H2), jnp.float32),
        ],
        compiler_params=pltpu.CompilerParams(
            dimension_semantics=("arbitrary", "arbitrary"),
        ),
    )(x, m, W1, b1.reshape(1, -1), g1.reshape(1, -1), be1.reshape(1, -1),
      W2, b2.reshape(1, -1), g2.reshape(1, -1), be2.reshape(1, -1),
      W3, b3.reshape(1, -1))
    return out


def kernel(bbox_ltwh, feats_masks, W1, b1, g1, be1, W2, b2, g2, be2, W3, b3):
    B, N, T, IN = bbox_ltwh.shape
    R = B * N
    x = bbox_ltwh[:, :, 0].reshape(R, IN)
    m = feats_masks[:, :, 0].reshape(R, 1).astype(jnp.float32)
    out = _fused_mlp(x, m, W1, b1, g1, be1, W2, b2, g2, be2, W3, b3, blk=2048)
    return out.reshape(B, N, W3.shape[1])


# speedup vs baseline: 1.0124x; 1.0124x over previous
import jax
import jax.numpy as jnp
from jax.experimental import pallas as pl
from jax.experimental.pallas import tpu as pltpu

_EPS = 1e-5


def _fused_mlp_kernel(x_ref, m_ref, W1_ref, b1_ref, g1_ref, be1_ref,
                      W2_ref, b2_ref, g2_ref, be2_ref, W3_ref, b3_ref,
                      out_ref,
                      s1_ref, q1_ref, sa1_ref, S_ref, cnt_ref,
                      sc1_ref, sh1_ref, sc2_ref, sh2_ref):
    phase = pl.program_id(0)
    i = pl.program_id(1)

    @pl.when((phase == 0) & (i == 0))
    def _init():
        s1_ref[...] = jnp.zeros_like(s1_ref)
        q1_ref[...] = jnp.zeros_like(q1_ref)
        sa1_ref[...] = jnp.zeros_like(sa1_ref)
        S_ref[...] = jnp.zeros_like(S_ref)
        cnt_ref[0, 0] = 0.0

    x = x_ref[...]
    m = m_ref[...]
    h1 = jnp.dot(x, W1_ref[...], preferred_element_type=jnp.float32) + b1_ref[...]

    @pl.when(phase == 0)
    def _p0():
        hm = h1 * m
        s1_ref[...] += jnp.sum(hm, axis=0, keepdims=True)
        q1_ref[...] += jnp.sum(hm * h1, axis=0, keepdims=True)
        cnt_ref[0, 0] += jnp.sum(m)

    @pl.when((phase == 1) & (i == 0))
    def _bn1_params():
        c = jnp.maximum(cnt_ref[0, 0], 1.0)
        mean = s1_ref[...] / c
        var = q1_ref[...] / c - mean * mean
        sc = g1_ref[...] * jax.lax.rsqrt(var + _EPS)
        sc1_ref[...] = sc
        sh1_ref[...] = be1_ref[...] - mean * sc

    @pl.when(phase >= 1)
    def _p12():
        a1 = jnp.maximum(h1 * sc1_ref[...] + sh1_ref[...], 0.0)

        @pl.when(phase == 1)
        def _p1():
            a1m = a1 * m
            sa1_ref[...] += jnp.sum(a1m, axis=0, keepdims=True)
            S_ref[...] += jax.lax.dot_general(
                a1m, a1, (((0,), (0,)), ((), ())),
                preferred_element_type=jnp.float32)

        @pl.when(phase == 2)
        def _p2():
            @pl.when(i == 0)
            def _bn2_params():
                c = jnp.maximum(cnt_ref[0, 0], 1.0)
                W2v = W2_ref[...]
                b2v = b2_ref[...]
                sh2v = jnp.dot(sa1_ref[...], W2v,
                               preferred_element_type=jnp.float32)
                T = jnp.dot(S_ref[...], W2v,
                            preferred_element_type=jnp.float32)
                q2 = (jnp.sum(W2v * T, axis=0, keepdims=True)
                      + 2.0 * b2v * sh2v + c * b2v * b2v)
                mean = sh2v / c + b2v
                var = q2 / c - mean * mean
                sc = g2_ref[...] * jax.lax.rsqrt(var + _EPS)
                sc2_ref[...] = sc
                sh2_ref[...] = be2_ref[...] - mean * sc

            h2 = jnp.dot(a1.astype(jnp.bfloat16), W2_ref[...].astype(jnp.bfloat16),
                         preferred_element_type=jnp.float32) + b2_ref[...]
            a2 = jnp.maximum(h2 * sc2_ref[...] + sh2_ref[...], 0.0)
            y = jnp.dot(a2.astype(jnp.bfloat16), W3_ref[...].astype(jnp.bfloat16),
                        preferred_element_type=jnp.float32) + b3_ref[...]
            out_ref[...] = y * m


def _fused_mlp(x, m, W1, b1, g1, be1, W2, b2, g2, be2, W3, b3, blk):
    R, IN = x.shape
    H1 = W1.shape[1]
    H2 = W2.shape[1]
    OUTD = W3.shape[1]
    nb = R // blk

    def rows(p, i):
        return (i, 0)

    def whole(p, i):
        return (0, 0)

    out = pl.pallas_call(
        _fused_mlp_kernel,
        grid=(3, nb),
        in_specs=[
            pl.BlockSpec((blk, IN), rows),
            pl.BlockSpec((blk, 1), rows),
            pl.BlockSpec((IN, H1), whole),
            pl.BlockSpec((1, H1), whole),
            pl.BlockSpec((1, H1), whole),
            pl.BlockSpec((1, H1), whole),
            pl.BlockSpec((H1, H2), whole),
            pl.BlockSpec((1, H2), whole),
            pl.BlockSpec((1, H2), whole),
            pl.BlockSpec((1, H2), whole),
            pl.BlockSpec((H2, OUTD), whole),
            pl.BlockSpec((1, OUTD), whole),
        ],
        out_specs=pl.BlockSpec((blk, OUTD), lambda p, i: (jnp.where(p == 2, i, 0), 0)),
        out_shape=jax.ShapeDtypeStruct((R, OUTD), jnp.float32),
        scratch_shapes=[
            pltpu.VMEM((1, H1), jnp.float32),
            pltpu.VMEM((1, H1), jnp.float32),
            pltpu.VMEM((1, H1), jnp.float32),
            pltpu.VMEM((H1, H1), jnp.float32),
            pltpu.SMEM((1, 1), jnp.float32),
            pltpu.VMEM((1, H1), jnp.float32),
            pltpu.VMEM((1, H1), jnp.float32),
            pltpu.VMEM((1, H2), jnp.float32),
            pltpu.VMEM((1, H2), jnp.float32),
        ],
        compiler_params=pltpu.CompilerParams(
            dimension_semantics=("arbitrary", "arbitrary"),
        ),
    )(x, m, W1, b1.reshape(1, -1), g1.reshape(1, -1), be1.reshape(1, -1),
      W2, b2.reshape(1, -1), g2.reshape(1, -1), be2.reshape(1, -1),
      W3, b3.reshape(1, -1))
    return out


def kernel(bbox_ltwh, feats_masks, W1, b1, g1, be1, W2, b2, g2, be2, W3, b3):
    B, N, T, IN = bbox_ltwh.shape
    R = B * N
    x = bbox_ltwh[:, :, 0].reshape(R, IN)
    m = feats_masks[:, :, 0].reshape(R, 1).astype(jnp.float32)
    out = _fused_mlp(x, m, W1, b1, g1, be1, W2, b2, g2, be2, W3, b3, blk=2048)
    return out.reshape(B, N, W3.shape[1])


# speedup vs baseline: 1.0734x; 1.0603x over previous
import jax
import jax.numpy as jnp
from jax.experimental import pallas as pl
from jax.experimental.pallas import tpu as pltpu

_EPS = 1e-5

_ROWDOT = (((0,), (0,)), ((), ()))


def _fused_mlp_kernel(x_ref, m_ref, W1_ref, b1_ref, g1_ref, be1_ref,
                      W2_ref, b2_ref, g2_ref, be2_ref, W3_ref, b3_ref,
                      out_ref,
                      sx_ref, Sxx_ref, sa1_ref, S_ref, cnt_ref,
                      sc1_ref, sh1_ref, sc2_ref, sh2_ref):
    phase = pl.program_id(0)
    i = pl.program_id(1)

    @pl.when((phase == 0) & (i == 0))
    def _init():
        sx_ref[...] = jnp.zeros_like(sx_ref)
        Sxx_ref[...] = jnp.zeros_like(Sxx_ref)
        sa1_ref[...] = jnp.zeros_like(sa1_ref)
        S_ref[...] = jnp.zeros_like(S_ref)
        cnt_ref[0, 0] = 0.0

    x = x_ref[...]
    m = m_ref[...]

    @pl.when(phase == 0)
    def _p0():
        xm = x * m
        sx_ref[...] += jax.lax.dot_general(
            m, x, _ROWDOT, preferred_element_type=jnp.float32)
        Sxx_ref[...] += jax.lax.dot_general(
            xm, x, _ROWDOT, preferred_element_type=jnp.float32)
        cnt_ref[0, 0] += jnp.sum(m)

    @pl.when((phase == 1) & (i == 0))
    def _bn1_params():
        c = jnp.maximum(cnt_ref[0, 0], 1.0)
        W1v = W1_ref[...]
        s1 = jnp.dot(sx_ref[...], W1v, preferred_element_type=jnp.float32)
        q1 = jnp.sum(W1v * jnp.dot(Sxx_ref[...], W1v,
                                   preferred_element_type=jnp.float32),
                     axis=0, keepdims=True)
        mean = s1 / c
        var = q1 / c - mean * mean
        sc = g1_ref[...] * jax.lax.rsqrt(var + _EPS)
        sc1_ref[...] = sc
        sh1_ref[...] = be1_ref[...] - mean * sc

    @pl.when(phase >= 1)
    def _p12():
        h1 = jnp.dot(x, W1_ref[...], preferred_element_type=jnp.float32)
        a1 = jnp.maximum(h1 * sc1_ref[...] + sh1_ref[...], 0.0)

        @pl.when(phase == 1)
        def _p1():
            a1m = a1 * m
            sa1_ref[...] += jax.lax.dot_general(
                m, a1, _ROWDOT, preferred_element_type=jnp.float32)
            S_ref[...] += jax.lax.dot_general(
                a1m, a1, _ROWDOT, preferred_element_type=jnp.float32)

        @pl.when(phase == 2)
        def _p2():
            @pl.when(i == 0)
            def _bn2_params():
                c = jnp.maximum(cnt_ref[0, 0], 1.0)
                W2v = W2_ref[...]
                s2 = jnp.dot(sa1_ref[...], W2v,
                             preferred_element_type=jnp.float32)
                q2 = jnp.sum(W2v * jnp.dot(S_ref[...], W2v,
                                           preferred_element_type=jnp.float32),
                             axis=0, keepdims=True)
                mean = s2 / c
                var = q2 / c - mean * mean
                sc = g2_ref[...] * jax.lax.rsqrt(var + _EPS)
                sc2_ref[...] = sc
                sh2_ref[...] = be2_ref[...] - mean * sc

            h2 = jnp.dot(a1, W2_ref[...], preferred_element_type=jnp.float32)
            a2 = jnp.maximum(h2 * sc2_ref[...] + sh2_ref[...], 0.0)
            y = jnp.dot(a2, W3_ref[...], preferred_element_type=jnp.float32) + b3_ref[...]
            out_ref[...] = y * m


def _fused_mlp(x, m, W1, b1, g1, be1, W2, b2, g2, be2, W3, b3, blk):
    R, IN = x.shape
    H1 = W1.shape[1]
    H2 = W2.shape[1]
    OUTD = W3.shape[1]
    nb = R // blk

    def rows(p, i):
        return (i, 0)

    def whole(p, i):
        return (0, 0)

    out = pl.pallas_call(
        _fused_mlp_kernel,
        grid=(3, nb),
        in_specs=[
            pl.BlockSpec((blk, IN), rows),
            pl.BlockSpec((blk, 1), rows),
            pl.BlockSpec((IN, H1), whole),
            pl.BlockSpec((1, H1), whole),
            pl.BlockSpec((1, H1), whole),
            pl.BlockSpec((1, H1), whole),
            pl.BlockSpec((H1, H2), whole),
            pl.BlockSpec((1, H2), whole),
            pl.BlockSpec((1, H2), whole),
            pl.BlockSpec((1, H2), whole),
            pl.BlockSpec((H2, OUTD), whole),
            pl.BlockSpec((1, OUTD), whole),
        ],
        out_specs=pl.BlockSpec((blk, OUTD), lambda p, i: (jnp.where(p == 2, i, 0), 0)),
        out_shape=jax.ShapeDtypeStruct((R, OUTD), jnp.float32),
        scratch_shapes=[
            pltpu.VMEM((1, IN), jnp.float32),
            pltpu.VMEM((IN, IN), jnp.float32),
            pltpu.VMEM((1, H1), jnp.float32),
            pltpu.VMEM((H1, H1), jnp.float32),
            pltpu.SMEM((1, 1), jnp.float32),
            pltpu.VMEM((1, H1), jnp.float32),
            pltpu.VMEM((1, H1), jnp.float32),
            pltpu.VMEM((1, H2), jnp.float32),
            pltpu.VMEM((1, H2), jnp.float32),
        ],
        compiler_params=pltpu.CompilerParams(
            dimension_semantics=("arbitrary", "arbitrary"),
        ),
    )(x, m, W1, b1.reshape(1, -1), g1.reshape(1, -1), be1.reshape(1, -1),
      W2, b2.reshape(1, -1), g2.reshape(1, -1), be2.reshape(1, -1),
      W3, b3.reshape(1, -1))
    return out


def kernel(bbox_ltwh, feats_masks, W1, b1, g1, be1, W2, b2, g2, be2, W3, b3):
    B, N, T, IN = bbox_ltwh.shape
    R = B * N
    x = bbox_ltwh[:, :, 0].reshape(R, IN)
    m = feats_masks[:, :, 0].reshape(R, 1).astype(jnp.float32)
    out = _fused_mlp(x, m, W1, b1, g1, be1, W2, b2, g2, be2, W3, b3, blk=2048)
    return out.reshape(B, N, W3.shape[1])
